# Initial kernel scaffold; baseline (speedup 1.0000x reference)
import jax
import jax.numpy as jnp
from jax.experimental import pallas as pl


def kernel(mu, domain_ids, W):
    raise NotImplementedError("write your pallas kernel here")


# speedup vs baseline: 2.1256x; 2.1256x over previous
import functools

import jax
import jax.numpy as jnp
from jax import lax
from jax.experimental import pallas as pl
from jax.experimental.pallas import tpu as pltpu
from jax.experimental.pallas import tpu_sc as plsc

B = 4096
DIM = 1024
ND = 8

NC = 2
NS = 16
NW = NC * NS
BPW = B // NW
CH = 32
NCH = BPW // CH

T = 256
NT = B // T
MAXU = NT + ND - 1

def _wid():
    return lax.axis_index("s") * NC + lax.axis_index("c")


@functools.lru_cache(maxsize=None)
def _sc_kernels():
    mesh = plsc.VectorSubcoreMesh(core_axis_name="c", subcore_axis_name="s")

    @functools.partial(
        pl.kernel,
        mesh=mesh,
        out_type=jax.ShapeDtypeStruct((B, DIM), jnp.float32),
        scratch_types=[
            pltpu.VMEM((NCH, CH), jnp.int32),
            pltpu.VMEM((CH, DIM), jnp.float32),
            pltpu.SemaphoreType.DMA,
        ],
    )
    def sc_gather(mu_hbm, idx_hbm, o_hbm, idx_v, buf, sem):
        base = _wid() * BPW
        pltpu.sync_copy(idx_hbm.at[_wid()], idx_v)
        for ch in range(NCH):
            pltpu.async_copy(mu_hbm.at[idx_v.at[ch]], buf, sem).wait()
            pltpu.sync_copy(buf, o_hbm.at[pl.ds(base + ch * CH, CH)])

    @functools.partial(
        pl.kernel,
        mesh=mesh,
        out_type=jax.ShapeDtypeStruct((B, DIM), jnp.float32),
        scratch_types=[
            pltpu.VMEM((NCH, CH), jnp.int32),
            pltpu.VMEM((CH, DIM), jnp.float32),
            pltpu.SemaphoreType.DMA,
        ],
    )
    def sc_scatter(ys_hbm, idx_hbm, o_hbm, idx_v, buf, sem):
        base = _wid() * BPW
        pltpu.sync_copy(idx_hbm.at[_wid()], idx_v)
        for ch in range(NCH):
            pltpu.sync_copy(ys_hbm.at[pl.ds(base + ch * CH, CH)], buf)
            pltpu.async_copy(buf, o_hbm.at[idx_v.at[ch]], sem).wait()

    return sc_gather, sc_scatter


def _mm_body(um_ref, ug_ref, lo_ref, hi_ref, xs_ref, w_ref, o_ref):
    u = pl.program_id(0)
    m = um_ref[u]
    prev_m = um_ref[jnp.maximum(u - 1, 0)]
    first = jnp.logical_or(u == 0, prev_m != m)

    @pl.when(first)
    def _():
        o_ref[...] = jnp.zeros_like(o_ref)

    rows = m * T + lax.broadcasted_iota(jnp.int32, (T, 1), 0)
    mask = jnp.logical_and(rows >= lo_ref[u], rows < hi_ref[u])
    xw = lax.dot_general(
        xs_ref[...], w_ref[0],
        (((1,), (1,)), ((), ())),
        preferred_element_type=jnp.float32,
    )
    o_ref[...] += jnp.where(mask, xw, 0.0)


def _grouped_matmul(um, ug, lo, hi, xs, W):
    grid_spec = pltpu.PrefetchScalarGridSpec(
        num_scalar_prefetch=4,
        grid=(MAXU,),
        in_specs=[
            pl.BlockSpec((T, DIM), lambda u, um, ug, lo, hi: (um[u], 0)),
            pl.BlockSpec((1, DIM, DIM), lambda u, um, ug, lo, hi: (ug[u], 0, 0)),
        ],
        out_specs=pl.BlockSpec((T, DIM), lambda u, um, ug, lo, hi: (um[u], 0)),
    )
    return pl.pallas_call(
        _mm_body,
        grid_spec=grid_spec,
        out_shape=jax.ShapeDtypeStruct((B, DIM), jnp.float32),
    )(um, ug, lo, hi, xs, W)


def _reg_body(w_ref, o_ref, acc_ref, ssq_ref):
    i = pl.program_id(0)
    w = w_ref[0]

    @pl.when(i == 0)
    def _():
        acc_ref[...] = jnp.zeros_like(acc_ref)
        ssq_ref[0] = 0.0

    acc_ref[...] += w
    ssq_ref[0] += jnp.sum(w * w)

    @pl.when(i == ND - 1)
    def _():
        a = acc_ref[...] * (1.0 / ND)
        o_ref[0, 0] = ssq_ref[0] * (1.0 / (ND * DIM * DIM)) - jnp.sum(a * a) * (
            1.0 / (DIM * DIM))


def _reg_loss(W):
    return pl.pallas_call(
        _reg_body,
        grid=(ND,),
        in_specs=[pl.BlockSpec((1, DIM, DIM), lambda i: (i, 0, 0))],
        out_specs=pl.BlockSpec((1, 1), lambda i: (0, 0), memory_space=pltpu.SMEM),
        out_shape=jax.ShapeDtypeStruct((1, 1), jnp.float32),
        scratch_shapes=[
            pltpu.VMEM((DIM, DIM), jnp.float32),
            pltpu.SMEM((1,), jnp.float32),
        ],
    )(W)


def _worklist(ids):
    counts = jnp.sum((ids[None, :] == jnp.arange(ND, dtype=jnp.int32)[:, None])
                     .astype(jnp.int32), axis=1)
    offs = jnp.concatenate([jnp.zeros((1,), jnp.int32), jnp.cumsum(counts)])
    fg = offs[:-1] // T
    lg = jnp.maximum(offs[1:] - 1, 0) // T
    ng = jnp.where(counts > 0, lg - fg + 1, 0)
    starts = jnp.concatenate([jnp.zeros((1,), jnp.int32),
                              jnp.cumsum(ng)[:-1]]).astype(jnp.int32)
    total = jnp.sum(ng)
    u = jnp.arange(MAXU, dtype=jnp.int32)
    valid = u < total
    uc = jnp.minimum(u, total - 1)
    g_of_u = (jnp.searchsorted(starts, uc, side="right") - 1).astype(jnp.int32)
    m_of_u = fg[g_of_u] + (uc - starts[g_of_u])
    lo = jnp.maximum(offs[g_of_u], m_of_u * T)
    hi = jnp.minimum(offs[g_of_u + 1], (m_of_u + 1) * T)
    lo = jnp.where(valid, lo, 0)
    hi = jnp.where(valid, hi, 0)
    return m_of_u.astype(jnp.int32), g_of_u, lo.astype(jnp.int32), hi.astype(jnp.int32)


def kernel(mu, domain_ids, W):
    ids = domain_ids.astype(jnp.int32)
    sort_idx = jnp.argsort(ids).astype(jnp.int32)
    um, ug, lo, hi = _worklist(ids)
    idx3 = sort_idx.reshape(NW, NCH, CH)

    sc_gather, sc_scatter = _sc_kernels()
    xs = sc_gather(mu, idx3)
    ys = _grouped_matmul(um, ug, lo, hi, xs, W)
    out = sc_scatter(ys, idx3)
    reg = _reg_loss(W)
    return out, reg[0, 0]
